# baseline (speedup 1.0000x reference)
import functools

import jax
import jax.numpy as jnp
from jax import lax
from jax.experimental import pallas as pl
from jax.experimental.pallas import tpu as pltpu
from jax.experimental.pallas import tpu_sc as plsc

NC = 2
NS = 16
CHUNK = 128
FH = 128
NBUF = 2
NP = 8
WIN = 1280
TRASH = WIN
GALIGN = NS * CHUNK * NBUF


@functools.cache
def _make_edge_agg(n, e_tot):
    assert n % 80 == 0 and n <= NP * WIN
    rpt8 = (n // NS) // 8 * 8
    last_len = n - (NS - 1) * rpt8
    mesh = plsc.VectorSubcoreMesh(
        core_axis_name="c", subcore_axis_name="s",
        num_cores=NC, num_subcores=NS)

    @functools.partial(
        pl.kernel,
        out_type=(jax.ShapeDtypeStruct((n, FH), jnp.float32),
                  jax.ShapeDtypeStruct((n, FH), jnp.float32)),
        mesh=mesh,
        scratch_types=[
            pltpu.VMEM((NBUF, CHUNK), jnp.int32),
            pltpu.VMEM((NBUF, CHUNK), jnp.int32),
            pltpu.VMEM((NBUF, CHUNK, FH), jnp.float32),
            pltpu.VMEM_SHARED((n, FH), jnp.float32),
            pltpu.VMEM_SHARED((WIN + 8, FH), jnp.float32),
            pltpu.VMEM((16,), jnp.int32),
        ] + [pltpu.SemaphoreType.DMA] * (3 * NBUF),
    )
    def k(h0_hbm, h1_hbm, srcp_hbm, dstl_hbm, offs_hbm, z0_hbm, z1_hbm,
          sidx, didx, gbuf, tbl, acc, offs_sm, *sems):
        isem = sems[:NBUF]
        gsem = sems[NBUF:2 * NBUF]
        ssem = sems[2 * NBUF:]
        c = lax.axis_index("c")
        s = lax.axis_index("s")
        pltpu.sync_copy(offs_hbm, offs_sm)
        ovec = offs_sm[...]

        row0 = s * rpt8
        for ci, h_hbm in enumerate((h0_hbm, h1_hbm)):
            @pl.when((c == ci) & (s < NS - 1))
            def _(h_hbm=h_hbm):
                pltpu.sync_copy(h_hbm.at[pl.ds(row0, rpt8)],
                                tbl.at[pl.ds(row0, rpt8)])

            @pl.when((c == ci) & (s == NS - 1))
            def _(h_hbm=h_hbm):
                pltpu.sync_copy(h_hbm.at[pl.ds((NS - 1) * rpt8, last_len)],
                                tbl.at[pl.ds((NS - 1) * rpt8, last_len)])
        plsc.subcore_barrier()

        def idx_load(cj, b):
            off = cj * CHUNK
            pltpu.async_copy(srcp_hbm.at[pl.ds(off, CHUNK)], sidx.at[b],
                             isem[b])
            pltpu.async_copy(dstl_hbm.at[pl.ds(off, CHUNK)], didx.at[b],
                             isem[b])

        def idx_wait(b):
            pltpu.make_async_copy(srcp_hbm.at[pl.ds(0, CHUNK)], sidx.at[b],
                                  isem[b]).wait()
            pltpu.make_async_copy(dstl_hbm.at[pl.ds(0, CHUNK)], didx.at[b],
                                  isem[b]).wait()

        def gather(b):
            pltpu.async_copy(tbl.at[sidx.at[b]], gbuf.at[b], gsem[b])

        def gather_wait(b):
            pltpu.make_async_copy(tbl.at[sidx.at[0]], gbuf.at[b],
                                  gsem[b]).wait()

        def scatter(b):
            pltpu.async_copy(gbuf.at[b], acc.at[didx.at[b]], ssem[b],
                             add=True)

        def scatter_wait(b):
            pltpu.make_async_copy(gbuf.at[b], acc.at[didx.at[0]],
                                  ssem[b]).wait()

        for p in range(NP):
            gbase = p * WIN + s * 80
            abase = s * 80
            for ci, h_hbm in enumerate((h0_hbm, h1_hbm)):
                @pl.when((c == ci) & (gbase + 80 <= n))
                def _(h_hbm=h_hbm):
                    pltpu.sync_copy(h_hbm.at[pl.ds(gbase, 80)],
                                    acc.at[pl.ds(abase, 80)])
            plsc.subcore_barrier()

            tchunks = (ovec[p + 1] - ovec[p]) // NS
            ch0 = ovec[p] + s * tchunks

            for b in range(NBUF):
                idx_load(ch0 + b, b)

            @pl.loop(0, tchunks - NBUF, step=NBUF)
            def _(j):
                for b in range(NBUF):
                    idx_wait(b)
                    gather(b)
                for b in range(NBUF):
                    gather_wait(b)
                    scatter(b)
                for b in range(NBUF):
                    scatter_wait(b)
                    idx_load(ch0 + j + NBUF + b, b)

            for b in range(NBUF):
                idx_wait(b)
                gather(b)
            for b in range(NBUF):
                gather_wait(b)
                scatter(b)
            for b in range(NBUF):
                scatter_wait(b)
            plsc.subcore_barrier()

            for ci, z_hbm in enumerate((z0_hbm, z1_hbm)):
                @pl.when((c == ci) & (gbase + 80 <= n))
                def _(z_hbm=z_hbm):
                    pltpu.sync_copy(acc.at[pl.ds(abase, 80)],
                                    z_hbm.at[pl.ds(gbase, 80)])

    return k


def _partition_edges(src, dst, n):
    e = src.shape[0]
    nbkt = ((e + GALIGN - 1) // GALIGN + NP) * GALIGN
    key = dst // WIN
    onehot = key[None, :] == jnp.arange(NP, dtype=jnp.int32)[:, None]
    cnt = jnp.sum(onehot, axis=1, dtype=jnp.int32)
    cnt_pad = jnp.maximum((cnt + GALIGN - 1) // GALIGN, 1) * GALIGN
    off_e = jnp.concatenate([jnp.zeros((1,), jnp.int32),
                             jnp.cumsum(cnt_pad, dtype=jnp.int32)])
    rank = jnp.sum(jnp.where(onehot, jnp.cumsum(onehot, axis=1,
                                                dtype=jnp.int32) - 1, 0),
                   axis=0)
    pos = off_e[key] + rank
    fill_src = (jnp.arange(nbkt, dtype=jnp.int32) * 37) % n
    fill_dst = TRASH + jnp.arange(nbkt, dtype=jnp.int32) % 8
    src_p = fill_src.at[pos].set(src)
    dst_l = fill_dst.at[pos].set(dst - key * WIN)
    offs = jnp.zeros((16,), jnp.int32).at[:NP + 1].set(off_e // CHUNK)
    return src_p, dst_l, offs, nbkt


@functools.cache
def _make_mlp(n, bm):
    def body(z0, z1, w1, b1, w2, b2, o0, o1):
        y = jnp.dot(z0[...], w1[:FH, :], preferred_element_type=jnp.float32)
        y += jnp.dot(z1[...], w1[FH:, :], preferred_element_type=jnp.float32)
        y = jnp.maximum(y + b1[...], 0.0)
        h = jnp.dot(y, w2[...], preferred_element_type=jnp.float32) + b2[...]
        h = jnp.maximum(h, 0.0)
        o0[...] = h[:, :FH]
        o1[...] = h[:, FH:]

    return pl.pallas_call(
        body,
        grid=(n // bm,),
        in_specs=[
            pl.BlockSpec((bm, FH), lambda i: (i, 0)),
            pl.BlockSpec((bm, FH), lambda i: (i, 0)),
            pl.BlockSpec((2 * FH, 2 * FH), lambda i: (0, 0)),
            pl.BlockSpec((1, 2 * FH), lambda i: (0, 0)),
            pl.BlockSpec((2 * FH, 2 * FH), lambda i: (0, 0)),
            pl.BlockSpec((1, 2 * FH), lambda i: (0, 0)),
        ],
        out_specs=[
            pl.BlockSpec((bm, FH), lambda i: (i, 0)),
            pl.BlockSpec((bm, FH), lambda i: (i, 0)),
        ],
        out_shape=(jax.ShapeDtypeStruct((n, FH), jnp.float32),
                   jax.ShapeDtypeStruct((n, FH), jnp.float32)),
    )


@functools.cache
def _make_pool_head(n, bm, g, d):
    nb = n // bm
    nh = 12

    def body(batch_ref, *refs):
        h_refs = refs[:nh]
        w1, b1, w2, b2, oh, ols, acc = refs[nh:]
        k = pl.program_id(0)

        @pl.when(k == 0)
        def _():
            acc[...] = jnp.zeros_like(acc)

        b = batch_ref[0, 0, :]
        onehot = (b[None, :] ==
                  lax.broadcasted_iota(jnp.int32, (g, bm), 0)
                  ).astype(jnp.float32)
        for idx, href in enumerate(h_refs):
            li, half = divmod(idx, 2)
            col = li * 2 * FH + half * FH
            acc[:, col:col + FH] += jnp.dot(
                onehot, href[...], preferred_element_type=jnp.float32)

        @pl.when(k == nb - 1)
        def _():
            p = acc[...]
            a1 = jnp.dot(p, w1[...], preferred_element_type=jnp.float32)
            a1 = jnp.maximum(a1 + b1[...], 0.0)
            z2 = jnp.dot(a1, w2[...], preferred_element_type=jnp.float32)
            z2 = z2 + b2[...]
            oh[...] = z2
            m = jnp.max(z2, axis=1, keepdims=True)
            ls = z2 - m
            ols[...] = ls - jnp.log(jnp.sum(jnp.exp(ls), axis=1,
                                            keepdims=True))

    h_spec = pl.BlockSpec((bm, FH), lambda i: (i, 0))
    return pl.pallas_call(
        body,
        grid=(nb,),
        in_specs=[pl.BlockSpec((1, 1, bm), lambda i: (i, 0, 0))]
        + [h_spec] * nh
        + [
            pl.BlockSpec((d, d), lambda i: (0, 0)),
            pl.BlockSpec((1, d), lambda i: (0, 0)),
            pl.BlockSpec((d, d), lambda i: (0, 0)),
            pl.BlockSpec((1, d), lambda i: (0, 0)),
        ],
        out_specs=[
            pl.BlockSpec((g, d), lambda i: (0, 0)),
            pl.BlockSpec((g, d), lambda i: (0, 0)),
        ],
        out_shape=(jax.ShapeDtypeStruct((g, d), jnp.float32),
                   jax.ShapeDtypeStruct((g, d), jnp.float32)),
        scratch_shapes=[pltpu.VMEM((g, d), jnp.float32)],
    )


def _fold_bn(p, f_in):
    scale = p["g"] / jnp.sqrt(1.0 + 1e-5)
    w1 = p["W1"] * scale[None, :]
    b1 = p["b1"] * scale + p["bb"]
    if f_in < 2 * FH:
        w1 = jnp.concatenate(
            [w1, jnp.zeros((2 * FH - f_in, w1.shape[1]), w1.dtype)], axis=0)
    return w1, b1.reshape(1, -1)


def kernel(x, edge_index, batch, params):
    n, f_in = x.shape
    g = 128
    bm = 1000

    src_p, dst_l, offs, e_tot = _partition_edges(
        edge_index[0], edge_index[1], n)

    edge_agg = _make_edge_agg(n, e_tot)
    mlp = _make_mlp(n, bm)

    h0 = x
    h1 = jnp.zeros((n, FH), jnp.float32)
    layer_params = [params["conv1"], params["conv2"], params["conv3"],
                    params["conv4"], params["conv4"], params["conv4"]]
    hs = []
    for li, p in enumerate(layer_params):
        z0, z1 = edge_agg(h0, h1, src_p, dst_l, offs)
        w1, b1 = _fold_bn(p, f_in if li == 0 else 2 * FH)
        h0, h1 = mlp(z0, z1, w1, b1, p["W2"], p["b2"].reshape(1, -1))
        hs.extend([h0, h1])

    d = 2 * FH * 6
    pool_head = _make_pool_head(n, bm, g, d)
    batch_r = batch.reshape(n // bm, 1, bm)
    oh, ols = pool_head(batch_r, *hs,
                        params["lin1"]["W"], params["lin1"]["b"].reshape(1, -1),
                        params["lin2"]["W"], params["lin2"]["b"].reshape(1, -1))
    return (oh, ols)


# speedup vs baseline: 1.3360x; 1.3360x over previous
import functools

import jax
import jax.numpy as jnp
from jax import lax
from jax.experimental import pallas as pl
from jax.experimental.pallas import tpu as pltpu
from jax.experimental.pallas import tpu_sc as plsc

NC = 2
NS = 16
CHUNK = 128
FH = 128
NBUF = 2
NP = 8
WIN = 1280
TRASH = WIN
GALIGN = NS * CHUNK * NBUF


@functools.cache
def _make_edge_agg(n, e_tot):
    assert n % 80 == 0 and n <= NP * WIN
    rpt8 = (n // NS) // 8 * 8
    last_len = n - (NS - 1) * rpt8
    mesh = plsc.VectorSubcoreMesh(
        core_axis_name="c", subcore_axis_name="s",
        num_cores=NC, num_subcores=NS)

    @functools.partial(
        pl.kernel,
        out_type=(jax.ShapeDtypeStruct((n, FH), jnp.float32),
                  jax.ShapeDtypeStruct((n, FH), jnp.float32)),
        mesh=mesh,
        scratch_types=[
            pltpu.VMEM((NBUF, CHUNK), jnp.int32),
            pltpu.VMEM((NBUF, CHUNK), jnp.int32),
            pltpu.VMEM((NBUF, CHUNK, FH), jnp.float32),
            pltpu.VMEM_SHARED((n, FH), jnp.float32),
            pltpu.VMEM_SHARED((WIN + 8, FH), jnp.float32),
            pltpu.VMEM((16,), jnp.int32),
        ] + [pltpu.SemaphoreType.DMA] * (3 * NBUF),
    )
    def k(h0_hbm, h1_hbm, srcp_hbm, dstl_hbm, offs_hbm, z0_hbm, z1_hbm,
          sidx, didx, gbuf, tbl, acc, offs_sm, *sems):
        isem = sems[:NBUF]
        gsem = sems[NBUF:2 * NBUF]
        ssem = sems[2 * NBUF:]
        c = lax.axis_index("c")
        s = lax.axis_index("s")
        pltpu.sync_copy(offs_hbm, offs_sm)
        ovec = offs_sm[...]

        row0 = s * rpt8
        for ci, h_hbm in enumerate((h0_hbm, h1_hbm)):
            @pl.when((c == ci) & (s < NS - 1))
            def _(h_hbm=h_hbm):
                pltpu.sync_copy(h_hbm.at[pl.ds(row0, rpt8)],
                                tbl.at[pl.ds(row0, rpt8)])

            @pl.when((c == ci) & (s == NS - 1))
            def _(h_hbm=h_hbm):
                pltpu.sync_copy(h_hbm.at[pl.ds((NS - 1) * rpt8, last_len)],
                                tbl.at[pl.ds((NS - 1) * rpt8, last_len)])
        plsc.subcore_barrier()

        def idx_load(cj, b):
            off = cj * CHUNK
            pltpu.async_copy(srcp_hbm.at[pl.ds(off, CHUNK)], sidx.at[b],
                             isem[b])
            pltpu.async_copy(dstl_hbm.at[pl.ds(off, CHUNK)], didx.at[b],
                             isem[b])

        def idx_wait(b):
            pltpu.make_async_copy(srcp_hbm.at[pl.ds(0, CHUNK)], sidx.at[b],
                                  isem[b]).wait()
            pltpu.make_async_copy(dstl_hbm.at[pl.ds(0, CHUNK)], didx.at[b],
                                  isem[b]).wait()

        def gather(b):
            pltpu.async_copy(tbl.at[sidx.at[b]], gbuf.at[b], gsem[b])

        def gather_wait(b):
            pltpu.make_async_copy(tbl.at[sidx.at[0]], gbuf.at[b],
                                  gsem[b]).wait()

        def scatter(b):
            pltpu.async_copy(gbuf.at[b], acc.at[didx.at[b]], ssem[b],
                             add=True)

        def scatter_wait(b):
            pltpu.make_async_copy(gbuf.at[b], acc.at[didx.at[0]],
                                  ssem[b]).wait()

        for p in range(NP):
            gbase = p * WIN + s * 80
            abase = s * 80
            for ci, h_hbm in enumerate((h0_hbm, h1_hbm)):
                @pl.when((c == ci) & (gbase + 80 <= n))
                def _(h_hbm=h_hbm):
                    pltpu.sync_copy(h_hbm.at[pl.ds(gbase, 80)],
                                    acc.at[pl.ds(abase, 80)])
            plsc.subcore_barrier()

            tchunks = (ovec[p + 1] - ovec[p]) // NS
            ch0 = ovec[p] + s * tchunks

            for b in range(NBUF):
                idx_load(ch0 + b, b)

            @pl.loop(0, tchunks - NBUF, step=NBUF)
            def _(j):
                for b in range(NBUF):
                    idx_wait(b)
                    gather(b)
                for b in range(NBUF):
                    gather_wait(b)
                    scatter(b)
                for b in range(NBUF):
                    scatter_wait(b)
                    idx_load(ch0 + j + NBUF + b, b)

            for b in range(NBUF):
                idx_wait(b)
                gather(b)
            for b in range(NBUF):
                gather_wait(b)
                scatter(b)
            for b in range(NBUF):
                scatter_wait(b)
            plsc.subcore_barrier()

            for ci, z_hbm in enumerate((z0_hbm, z1_hbm)):
                @pl.when((c == ci) & (gbase + 80 <= n))
                def _(z_hbm=z_hbm):
                    pltpu.sync_copy(acc.at[pl.ds(abase, 80)],
                                    z_hbm.at[pl.ds(gbase, 80)])

    return k


def _partition_edges(src, dst, n):
    e = src.shape[0]
    nbkt = ((e + GALIGN - 1) // GALIGN + NP) * GALIGN
    key = dst // WIN
    onehot = key[None, :] == jnp.arange(NP, dtype=jnp.int32)[:, None]
    cnt = jnp.sum(onehot, axis=1, dtype=jnp.int32)
    cnt_pad = jnp.maximum((cnt + GALIGN - 1) // GALIGN, 1) * GALIGN
    off_e = jnp.concatenate([jnp.zeros((1,), jnp.int32),
                             jnp.cumsum(cnt_pad, dtype=jnp.int32)])
    rank = jnp.sum(jnp.where(onehot, jnp.cumsum(onehot, axis=1,
                                                dtype=jnp.int32) - 1, 0),
                   axis=0)
    pos = off_e[key] + rank
    ar = jnp.arange(nbkt, dtype=jnp.int32)
    fill = ((ar * 37) % n) * 2048 + TRASH + ar % 8
    packed = src * 2048 + (dst - key * WIN)
    arr = fill.at[pos].set(packed, unique_indices=True,
                           mode="promise_in_bounds")
    src_p = arr >> 11
    dst_l = arr & 2047
    offs = jnp.zeros((16,), jnp.int32).at[:NP + 1].set(off_e // CHUNK)
    return src_p, dst_l, offs, nbkt


@functools.cache
def _make_mlp(n, bm):
    def body(z0, z1, w1, b1, w2, b2, o0, o1):
        y = jnp.dot(z0[...], w1[:FH, :], preferred_element_type=jnp.float32)
        y += jnp.dot(z1[...], w1[FH:, :], preferred_element_type=jnp.float32)
        y = jnp.maximum(y + b1[...], 0.0)
        h = jnp.dot(y, w2[...], preferred_element_type=jnp.float32) + b2[...]
        h = jnp.maximum(h, 0.0)
        o0[...] = h[:, :FH]
        o1[...] = h[:, FH:]

    return pl.pallas_call(
        body,
        grid=(n // bm,),
        in_specs=[
            pl.BlockSpec((bm, FH), lambda i: (i, 0)),
            pl.BlockSpec((bm, FH), lambda i: (i, 0)),
            pl.BlockSpec((2 * FH, 2 * FH), lambda i: (0, 0)),
            pl.BlockSpec((1, 2 * FH), lambda i: (0, 0)),
            pl.BlockSpec((2 * FH, 2 * FH), lambda i: (0, 0)),
            pl.BlockSpec((1, 2 * FH), lambda i: (0, 0)),
        ],
        out_specs=[
            pl.BlockSpec((bm, FH), lambda i: (i, 0)),
            pl.BlockSpec((bm, FH), lambda i: (i, 0)),
        ],
        out_shape=(jax.ShapeDtypeStruct((n, FH), jnp.float32),
                   jax.ShapeDtypeStruct((n, FH), jnp.float32)),
    )


@functools.cache
def _make_pool_head(n, bm, g, d):
    nb = n // bm
    nh = 12

    def body(batch_ref, *refs):
        h_refs = refs[:nh]
        w1, b1, w2, b2, oh, ols, acc = refs[nh:]
        k = pl.program_id(0)

        @pl.when(k == 0)
        def _():
            acc[...] = jnp.zeros_like(acc)

        b = batch_ref[0, 0, :]
        onehot = (b[None, :] ==
                  lax.broadcasted_iota(jnp.int32, (g, bm), 0)
                  ).astype(jnp.float32)
        for idx, href in enumerate(h_refs):
            li, half = divmod(idx, 2)
            col = li * 2 * FH + half * FH
            acc[:, col:col + FH] += jnp.dot(
                onehot, href[...], preferred_element_type=jnp.float32)

        @pl.when(k == nb - 1)
        def _():
            p = acc[...]
            a1 = jnp.dot(p, w1[...], preferred_element_type=jnp.float32)
            a1 = jnp.maximum(a1 + b1[...], 0.0)
            z2 = jnp.dot(a1, w2[...], preferred_element_type=jnp.float32)
            z2 = z2 + b2[...]
            oh[...] = z2
            m = jnp.max(z2, axis=1, keepdims=True)
            ls = z2 - m
            ols[...] = ls - jnp.log(jnp.sum(jnp.exp(ls), axis=1,
                                            keepdims=True))

    h_spec = pl.BlockSpec((bm, FH), lambda i: (i, 0))
    return pl.pallas_call(
        body,
        grid=(nb,),
        in_specs=[pl.BlockSpec((1, 1, bm), lambda i: (i, 0, 0))]
        + [h_spec] * nh
        + [
            pl.BlockSpec((d, d), lambda i: (0, 0)),
            pl.BlockSpec((1, d), lambda i: (0, 0)),
            pl.BlockSpec((d, d), lambda i: (0, 0)),
            pl.BlockSpec((1, d), lambda i: (0, 0)),
        ],
        out_specs=[
            pl.BlockSpec((g, d), lambda i: (0, 0)),
            pl.BlockSpec((g, d), lambda i: (0, 0)),
        ],
        out_shape=(jax.ShapeDtypeStruct((g, d), jnp.float32),
                   jax.ShapeDtypeStruct((g, d), jnp.float32)),
        scratch_shapes=[pltpu.VMEM((g, d), jnp.float32)],
    )


def _fold_bn(p, f_in):
    scale = p["g"] / jnp.sqrt(1.0 + 1e-5)
    w1 = p["W1"] * scale[None, :]
    b1 = p["b1"] * scale + p["bb"]
    if f_in < 2 * FH:
        w1 = jnp.concatenate(
            [w1, jnp.zeros((2 * FH - f_in, w1.shape[1]), w1.dtype)], axis=0)
    return w1, b1.reshape(1, -1)


def kernel(x, edge_index, batch, params):
    n, f_in = x.shape
    g = 128
    bm = 1000

    src_p, dst_l, offs, e_tot = _partition_edges(
        edge_index[0], edge_index[1], n)

    edge_agg = _make_edge_agg(n, e_tot)
    mlp = _make_mlp(n, bm)

    h0 = x
    h1 = jnp.zeros((n, FH), jnp.float32)
    layer_params = [params["conv1"], params["conv2"], params["conv3"],
                    params["conv4"], params["conv4"], params["conv4"]]
    hs = []
    for li, p in enumerate(layer_params):
        z0, z1 = edge_agg(h0, h1, src_p, dst_l, offs)
        w1, b1 = _fold_bn(p, f_in if li == 0 else 2 * FH)
        h0, h1 = mlp(z0, z1, w1, b1, p["W2"], p["b2"].reshape(1, -1))
        hs.extend([h0, h1])

    d = 2 * FH * 6
    pool_head = _make_pool_head(n, bm, g, d)
    batch_r = batch.reshape(n // bm, 1, bm)
    oh, ols = pool_head(batch_r, *hs,
                        params["lin1"]["W"], params["lin1"]["b"].reshape(1, -1),
                        params["lin2"]["W"], params["lin2"]["b"].reshape(1, -1))
    return (oh, ols)


# speedup vs baseline: 1.7117x; 1.2812x over previous
import functools

import jax
import jax.numpy as jnp
from jax import lax
from jax.experimental import pallas as pl
from jax.experimental.pallas import tpu as pltpu
from jax.experimental.pallas import tpu_sc as plsc

NC = 2
NS = 16
CHUNK = 128
FH = 128
NBUF = 2
NP = 8
WIN = 1280
TRASH = WIN
GALIGN = NS * CHUNK * NBUF


@functools.cache
def _make_edge_agg(n, e_tot):
    assert n % 80 == 0 and n <= NP * WIN
    rpt8 = (n // NS) // 8 * 8
    last_len = n - (NS - 1) * rpt8
    mesh = plsc.VectorSubcoreMesh(
        core_axis_name="c", subcore_axis_name="s",
        num_cores=NC, num_subcores=NS)

    @functools.partial(
        pl.kernel,
        out_type=(jax.ShapeDtypeStruct((n, FH), jnp.float32),
                  jax.ShapeDtypeStruct((n, FH), jnp.float32)),
        mesh=mesh,
        scratch_types=[
            pltpu.VMEM((NBUF, CHUNK), jnp.int32),
            pltpu.VMEM((NBUF, CHUNK), jnp.int32),
            pltpu.VMEM((NBUF, CHUNK, FH), jnp.float32),
            pltpu.VMEM_SHARED((n, FH), jnp.float32),
            pltpu.VMEM_SHARED((WIN + 8, FH), jnp.float32),
            pltpu.VMEM((16,), jnp.int32),
        ] + [pltpu.SemaphoreType.DMA] * (3 * NBUF),
    )
    def k(h0_hbm, h1_hbm, srcp_hbm, dstl_hbm, offs_hbm, z0_hbm, z1_hbm,
          sidx, didx, gbuf, tbl, acc, offs_sm, *sems):
        isem = sems[:NBUF]
        gsem = sems[NBUF:2 * NBUF]
        ssem = sems[2 * NBUF:]
        c = lax.axis_index("c")
        s = lax.axis_index("s")
        pltpu.sync_copy(offs_hbm, offs_sm)
        ovec = offs_sm[...]

        row0 = s * rpt8
        for ci, h_hbm in enumerate((h0_hbm, h1_hbm)):
            @pl.when((c == ci) & (s < NS - 1))
            def _(h_hbm=h_hbm):
                pltpu.sync_copy(h_hbm.at[pl.ds(row0, rpt8)],
                                tbl.at[pl.ds(row0, rpt8)])

            @pl.when((c == ci) & (s == NS - 1))
            def _(h_hbm=h_hbm):
                pltpu.sync_copy(h_hbm.at[pl.ds((NS - 1) * rpt8, last_len)],
                                tbl.at[pl.ds((NS - 1) * rpt8, last_len)])
        plsc.subcore_barrier()

        ec = e_tot // CHUNK

        def idx_load(cj, b):
            off = jnp.minimum(cj, ec - 1) * CHUNK
            pltpu.async_copy(srcp_hbm.at[pl.ds(off, CHUNK)], sidx.at[b],
                             isem[b])
            pltpu.async_copy(dstl_hbm.at[pl.ds(off, CHUNK)], didx.at[b],
                             isem[b])

        def idx_wait(b):
            pltpu.make_async_copy(srcp_hbm.at[pl.ds(0, CHUNK)], sidx.at[b],
                                  isem[b]).wait()
            pltpu.make_async_copy(dstl_hbm.at[pl.ds(0, CHUNK)], didx.at[b],
                                  isem[b]).wait()

        def gather(b):
            pltpu.async_copy(tbl.at[sidx.at[b]], gbuf.at[b], gsem[b])

        def gather_wait(b):
            pltpu.make_async_copy(tbl.at[sidx.at[0]], gbuf.at[b],
                                  gsem[b]).wait()

        def scatter(b):
            pltpu.async_copy(gbuf.at[b], acc.at[didx.at[b]], ssem[b],
                             add=True)

        def scatter_wait(b):
            pltpu.make_async_copy(gbuf.at[b], acc.at[didx.at[0]],
                                  ssem[b]).wait()

        for p in range(NP):
            gbase = p * WIN + s * 80
            abase = s * 80
            for ci, h_hbm in enumerate((h0_hbm, h1_hbm)):
                @pl.when((c == ci) & (gbase + 80 <= n))
                def _(h_hbm=h_hbm):
                    pltpu.sync_copy(h_hbm.at[pl.ds(gbase, 80)],
                                    acc.at[pl.ds(abase, 80)])
            plsc.subcore_barrier()

            lo = p * WIN
            clo = ovec[p] // CHUNK
            chi = (ovec[p + 1] + CHUNK - 1) // CHUNK
            tch = (chi - clo + NS - 1) // NS
            tch = jnp.maximum((tch + NBUF - 1) // NBUF * NBUF, NBUF)
            ch0 = clo + s * tch

            def localize(b):
                for kq in range(CHUNK // 16):
                    d = didx[b, pl.ds(kq * 16, 16)]
                    m = (d >= lo) & (d < lo + WIN)
                    trash = TRASH + (
                        lax.broadcasted_iota(jnp.int32, (16,), 0) & 7)
                    didx[b, pl.ds(kq * 16, 16)] = jnp.where(m, d - lo, trash)

            for b in range(NBUF):
                idx_load(ch0 + b, b)

            @pl.loop(0, tch - NBUF, step=NBUF)
            def _(j):
                for b in range(NBUF):
                    idx_wait(b)
                    localize(b)
                    gather(b)
                for b in range(NBUF):
                    gather_wait(b)
                    scatter(b)
                for b in range(NBUF):
                    scatter_wait(b)
                    idx_load(ch0 + j + NBUF + b, b)

            for b in range(NBUF):
                idx_wait(b)
                localize(b)
                gather(b)
            for b in range(NBUF):
                gather_wait(b)
                scatter(b)
            for b in range(NBUF):
                scatter_wait(b)
            plsc.subcore_barrier()

            for ci, z_hbm in enumerate((z0_hbm, z1_hbm)):
                @pl.when((c == ci) & (gbase + 80 <= n))
                def _(z_hbm=z_hbm):
                    pltpu.sync_copy(acc.at[pl.ds(abase, 80)],
                                    z_hbm.at[pl.ds(gbase, 80)])

    return k


def _partition_edges(src, dst, n):
    e = src.shape[0]
    e_tot = ((e + CHUNK - 1) // CHUNK + 1) * CHUNK
    pad = e_tot - e
    ar = jnp.arange(pad, dtype=jnp.int32)
    dst_pad = jnp.concatenate([dst, n + ar % 200])
    src_pad = jnp.concatenate([src, (ar * 37) % n])
    sdst, ssrc = jax.lax.sort_key_val(dst_pad, src_pad)
    bounds = jnp.arange(NP + 1, dtype=jnp.int32) * WIN
    off_e = jnp.searchsorted(sdst, bounds, side="left").astype(jnp.int32)
    offs = jnp.zeros((16,), jnp.int32).at[:NP + 1].set(off_e)
    return ssrc, sdst, offs, e_tot


@functools.cache
def _make_mlp(n, bm):
    def body(z0, z1, w1, b1, w2, b2, o0, o1):
        y = jnp.dot(z0[...], w1[:FH, :], preferred_element_type=jnp.float32)
        y += jnp.dot(z1[...], w1[FH:, :], preferred_element_type=jnp.float32)
        y = jnp.maximum(y + b1[...], 0.0)
        h = jnp.dot(y, w2[...], preferred_element_type=jnp.float32) + b2[...]
        h = jnp.maximum(h, 0.0)
        o0[...] = h[:, :FH]
        o1[...] = h[:, FH:]

    return pl.pallas_call(
        body,
        grid=(n // bm,),
        in_specs=[
            pl.BlockSpec((bm, FH), lambda i: (i, 0)),
            pl.BlockSpec((bm, FH), lambda i: (i, 0)),
            pl.BlockSpec((2 * FH, 2 * FH), lambda i: (0, 0)),
            pl.BlockSpec((1, 2 * FH), lambda i: (0, 0)),
            pl.BlockSpec((2 * FH, 2 * FH), lambda i: (0, 0)),
            pl.BlockSpec((1, 2 * FH), lambda i: (0, 0)),
        ],
        out_specs=[
            pl.BlockSpec((bm, FH), lambda i: (i, 0)),
            pl.BlockSpec((bm, FH), lambda i: (i, 0)),
        ],
        out_shape=(jax.ShapeDtypeStruct((n, FH), jnp.float32),
                   jax.ShapeDtypeStruct((n, FH), jnp.float32)),
    )


@functools.cache
def _make_pool_head(n, bm, g, d):
    nb = n // bm
    nh = 12

    def body(batch_ref, *refs):
        h_refs = refs[:nh]
        w1, b1, w2, b2, oh, ols, acc = refs[nh:]
        k = pl.program_id(0)

        @pl.when(k == 0)
        def _():
            acc[...] = jnp.zeros_like(acc)

        b = batch_ref[0, 0, :]
        onehot = (b[None, :] ==
                  lax.broadcasted_iota(jnp.int32, (g, bm), 0)
                  ).astype(jnp.float32)
        for idx, href in enumerate(h_refs):
            li, half = divmod(idx, 2)
            col = li * 2 * FH + half * FH
            acc[:, col:col + FH] += jnp.dot(
                onehot, href[...], preferred_element_type=jnp.float32)

        @pl.when(k == nb - 1)
        def _():
            p = acc[...]
            a1 = jnp.dot(p, w1[...], preferred_element_type=jnp.float32)
            a1 = jnp.maximum(a1 + b1[...], 0.0)
            z2 = jnp.dot(a1, w2[...], preferred_element_type=jnp.float32)
            z2 = z2 + b2[...]
            oh[...] = z2
            m = jnp.max(z2, axis=1, keepdims=True)
            ls = z2 - m
            ols[...] = ls - jnp.log(jnp.sum(jnp.exp(ls), axis=1,
                                            keepdims=True))

    h_spec = pl.BlockSpec((bm, FH), lambda i: (i, 0))
    return pl.pallas_call(
        body,
        grid=(nb,),
        in_specs=[pl.BlockSpec((1, 1, bm), lambda i: (i, 0, 0))]
        + [h_spec] * nh
        + [
            pl.BlockSpec((d, d), lambda i: (0, 0)),
            pl.BlockSpec((1, d), lambda i: (0, 0)),
            pl.BlockSpec((d, d), lambda i: (0, 0)),
            pl.BlockSpec((1, d), lambda i: (0, 0)),
        ],
        out_specs=[
            pl.BlockSpec((g, d), lambda i: (0, 0)),
            pl.BlockSpec((g, d), lambda i: (0, 0)),
        ],
        out_shape=(jax.ShapeDtypeStruct((g, d), jnp.float32),
                   jax.ShapeDtypeStruct((g, d), jnp.float32)),
        scratch_shapes=[pltpu.VMEM((g, d), jnp.float32)],
    )


def _fold_bn(p, f_in):
    scale = p["g"] / jnp.sqrt(1.0 + 1e-5)
    w1 = p["W1"] * scale[None, :]
    b1 = p["b1"] * scale + p["bb"]
    if f_in < 2 * FH:
        w1 = jnp.concatenate(
            [w1, jnp.zeros((2 * FH - f_in, w1.shape[1]), w1.dtype)], axis=0)
    return w1, b1.reshape(1, -1)


def kernel(x, edge_index, batch, params):
    n, f_in = x.shape
    g = 128
    bm = 1000

    src_p, dst_l, offs, e_tot = _partition_edges(
        edge_index[0], edge_index[1], n)

    edge_agg = _make_edge_agg(n, e_tot)
    mlp = _make_mlp(n, bm)

    h0 = x
    h1 = jnp.zeros((n, FH), jnp.float32)
    layer_params = [params["conv1"], params["conv2"], params["conv3"],
                    params["conv4"], params["conv4"], params["conv4"]]
    hs = []
    for li, p in enumerate(layer_params):
        z0, z1 = edge_agg(h0, h1, src_p, dst_l, offs)
        w1, b1 = _fold_bn(p, f_in if li == 0 else 2 * FH)
        h0, h1 = mlp(z0, z1, w1, b1, p["W2"], p["b2"].reshape(1, -1))
        hs.extend([h0, h1])

    d = 2 * FH * 6
    pool_head = _make_pool_head(n, bm, g, d)
    batch_r = batch.reshape(n // bm, 1, bm)
    oh, ols = pool_head(batch_r, *hs,
                        params["lin1"]["W"], params["lin1"]["b"].reshape(1, -1),
                        params["lin2"]["W"], params["lin2"]["b"].reshape(1, -1))
    return (oh, ols)
